# baseline (speedup 1.0000x reference)
import functools

import jax
import jax.numpy as jnp
from jax import lax
from jax.experimental import pallas as pl
from jax.experimental.pallas import tpu as pltpu
from jax.experimental.pallas import tpu_sc as plsc

N = 10000
E = 320000
DF = 128
DH = 64

NC, NS = 2, 16
NW = NC * NS
CHUNK = 128
NBUF = 5
CPW = 80
GROUPS = CPW // NBUF
EPW = CPW * CHUNK
EP = NW * EPW
ROWS = 10112
RPT = ROWS // NS

W1 = 80
W2 = 64

RB = 2000
GRID = N // RB


def _make_sc_agg(width):
    mesh = plsc.VectorSubcoreMesh(
        core_axis_name="c", subcore_axis_name="s", num_cores=NC, num_subcores=NS
    )

    @functools.partial(
        pl.kernel,
        out_type=jax.ShapeDtypeStruct((NC, ROWS, width), jnp.float32),
        mesh=mesh,
        scratch_types=[
            pltpu.VMEM((CPW, CHUNK), jnp.int32),
            pltpu.VMEM((CPW, CHUNK), jnp.int32),
            pltpu.VMEM((NBUF, CHUNK, width), jnp.float32),
            pltpu.VMEM_SHARED((ROWS, width), jnp.float32),
            pltpu.SemaphoreType.DMA,
            pltpu.SemaphoreType.DMA,
        ],
        compiler_params=pltpu.CompilerParams(use_tc_tiling_on_sc=False),
    )
    def sc_agg(table_hbm, src_hbm, dst_hbm, zeros_hbm, out_hbm,
               src_v, dst_v, rows_v, acc_sh, sem_g, sem_s):
        cid = lax.axis_index("c")
        sid = lax.axis_index("s")
        wid = sid * NC + cid
        row0 = sid * RPT

        pltpu.sync_copy(zeros_hbm.at[pl.ds(row0, RPT)], acc_sh.at[pl.ds(row0, RPT)])
        pltpu.sync_copy(src_hbm.at[wid], src_v)
        pltpu.sync_copy(dst_hbm.at[wid], dst_v)
        plsc.subcore_barrier()

        def body(g, carry):
            base = g * NBUF
            gd = [pltpu.async_copy(table_hbm.at[src_v.at[base + b]],
                                   rows_v.at[b], sem_g)
                  for b in range(NBUF)]
            for d in gd:
                d.wait()
            sd = [pltpu.async_copy(rows_v.at[b], acc_sh.at[dst_v.at[base + b]],
                                   sem_s, add=True)
                  for b in range(NBUF)]
            for d in sd:
                d.wait()
            return carry
        lax.fori_loop(0, GROUPS, body, 0)

        plsc.subcore_barrier()
        pltpu.sync_copy(acc_sh.at[pl.ds(row0, RPT)], out_hbm.at[cid, pl.ds(row0, RPT)])

    return sc_agg


_sc_agg80 = _make_sc_agg(W1)
_sc_agg64 = _make_sc_agg(W2)


def _t1_body(x_ref, w_ref, o_ref):
    y = jnp.dot(x_ref[...], w_ref[...], preferred_element_type=jnp.float32)
    col = lax.broadcasted_iota(jnp.int32, (RB, W1 - DH), 1)
    pad = jnp.where(col == 0, jnp.float32(1), jnp.float32(0))
    o_ref[...] = jnp.concatenate([y, pad], axis=1)


_t1 = pl.pallas_call(
    _t1_body,
    grid=(GRID,),
    in_specs=[
        pl.BlockSpec((RB, DF), lambda i: (i, 0)),
        pl.BlockSpec((DF, DH), lambda i: (0, 0)),
    ],
    out_specs=pl.BlockSpec((RB, W1), lambda i: (i, 0)),
    out_shape=jax.ShapeDtypeStruct((N, W1), jnp.float32),
)


def _c_body(acc_ref, x_ref, w1r_ref, b1_ref, wlin_ref, blin_ref,
            w2l_ref, w2r_ref, b2_ref, y2_ref, z2_ref, cnt_ref):
    acc = acc_ref[...]
    feat = acc[0, :, 0:DH] + acc[1, :, 0:DH]
    cnt = jnp.maximum(acc[0, :, DH] + acc[1, :, DH], 1.0)
    agg = feat / cnt[:, None]
    xb = x_ref[...]
    pre = agg + jnp.dot(xb, w1r_ref[...], preferred_element_type=jnp.float32) + b1_ref[...]
    nrm = jnp.sqrt(jnp.sum(pre * pre, axis=1, keepdims=True))
    hidden = jnp.maximum(pre / jnp.maximum(nrm, 1e-12), 0.0)
    wl = wlin_ref[...]
    h = jnp.maximum(
        jnp.dot(xb, wl[0:DF], preferred_element_type=jnp.float32)
        + jnp.dot(hidden, wl[DF:DF + DH], preferred_element_type=jnp.float32)
        + blin_ref[...],
        0.0,
    )
    y2_ref[...] = jnp.dot(h, w2l_ref[...], preferred_element_type=jnp.float32)
    z2_ref[...] = jnp.dot(h, w2r_ref[...], preferred_element_type=jnp.float32) + b2_ref[...]
    cnt_ref[...] = jnp.broadcast_to(cnt[:, None], (RB, 8))


_combine1 = pl.pallas_call(
    _c_body,
    grid=(GRID,),
    in_specs=[
        pl.BlockSpec((NC, RB, W1), lambda i: (0, i, 0)),
        pl.BlockSpec((RB, DF), lambda i: (i, 0)),
        pl.BlockSpec((DF, DH), lambda i: (0, 0)),
        pl.BlockSpec((1, DH), lambda i: (0, 0)),
        pl.BlockSpec((DF + DH, DH), lambda i: (0, 0)),
        pl.BlockSpec((1, DH), lambda i: (0, 0)),
        pl.BlockSpec((DH, DH), lambda i: (0, 0)),
        pl.BlockSpec((DH, DH), lambda i: (0, 0)),
        pl.BlockSpec((1, DH), lambda i: (0, 0)),
    ],
    out_specs=[
        pl.BlockSpec((RB, DH), lambda i: (i, 0)),
        pl.BlockSpec((RB, DH), lambda i: (i, 0)),
        pl.BlockSpec((RB, 8), lambda i: (i, 0)),
    ],
    out_shape=[
        jax.ShapeDtypeStruct((N, DH), jnp.float32),
        jax.ShapeDtypeStruct((N, DH), jnp.float32),
        jax.ShapeDtypeStruct((N, 8), jnp.float32),
    ],
)


def _e_body(acc_ref, z2_ref, cnt_ref, o_ref):
    acc = acc_ref[...]
    feat = acc[0] + acc[1]
    cnt = cnt_ref[...][:, 0]
    o = feat / cnt[:, None] + z2_ref[...]
    nrm = jnp.sqrt(jnp.sum(o * o, axis=1, keepdims=True))
    o_ref[...] = o / jnp.maximum(nrm, 1e-12)


_combine2 = pl.pallas_call(
    _e_body,
    grid=(GRID,),
    in_specs=[
        pl.BlockSpec((NC, RB, W2), lambda i: (0, i, 0)),
        pl.BlockSpec((RB, DH), lambda i: (i, 0)),
        pl.BlockSpec((RB, 8), lambda i: (i, 0)),
    ],
    out_specs=pl.BlockSpec((RB, DH), lambda i: (i, 0)),
    out_shape=jax.ShapeDtypeStruct((N, DH), jnp.float32),
)


def kernel(x, edge_index, W1_l, b1_l, W1_r, W_lin, b_lin, W2_l, b2_l, W2_r):
    ei = edge_index.astype(jnp.int32)
    ppw = EPW - E // NW
    pad_dst = jnp.broadcast_to(N + jnp.arange(ppw, dtype=jnp.int32)[None, :], (NW, ppw))
    src = jnp.pad(ei[0].reshape(NW, E // NW), ((0, 0), (0, ppw))).reshape(NW, CPW, CHUNK)
    dst = jnp.concatenate([ei[1].reshape(NW, E // NW), pad_dst], axis=1).reshape(NW, CPW, CHUNK)

    table1 = _t1(x, W1_l)
    acc1 = _sc_agg80(table1, src, dst, jnp.zeros((ROWS, W1), jnp.float32))
    y2, z2, cnt8 = _combine1(
        acc1, x, W1_r, b1_l.reshape(1, DH), W_lin, b_lin.reshape(1, DH),
        W2_l, W2_r, b2_l.reshape(1, DH),
    )
    acc2 = _sc_agg64(y2, src, dst, jnp.zeros((ROWS, W2), jnp.float32))
    return _combine2(acc2, z2, cnt8)


# speedup vs baseline: 1.5807x; 1.5807x over previous
import functools

import jax
import jax.numpy as jnp
from jax import lax
from jax.experimental import pallas as pl
from jax.experimental.pallas import tpu as pltpu
from jax.experimental.pallas import tpu_sc as plsc

N = 10000
E = 320000
DF = 128
DH = 64

NC, NS = 2, 16
NW = NC * NS
CHUNK = 128
NBUF = 3
CPW = 79
EPW = CPW * CHUNK
EP = NW * EPW
ROWS = 10112
RPT = ROWS // NS

W1 = 80
W2 = 64

RB = 2000
GRID = N // RB


def _make_sc_agg(width):
    mesh = plsc.VectorSubcoreMesh(
        core_axis_name="c", subcore_axis_name="s", num_cores=NC, num_subcores=NS
    )

    @functools.partial(
        pl.kernel,
        out_type=jax.ShapeDtypeStruct((NC, ROWS, width), jnp.float32),
        mesh=mesh,
        scratch_types=[
            pltpu.VMEM((CPW, CHUNK), jnp.int32),
            pltpu.VMEM((CPW, CHUNK), jnp.int32),
            pltpu.VMEM((NBUF, CHUNK, width), jnp.float32),
            pltpu.VMEM_SHARED((ROWS, width), jnp.float32),
            pltpu.SemaphoreType.DMA,
            pltpu.SemaphoreType.DMA,
        ],
        compiler_params=pltpu.CompilerParams(use_tc_tiling_on_sc=False),
    )
    def sc_agg(table_hbm, src_hbm, dst_hbm, zeros_hbm, out_hbm,
               src_v, dst_v, rows_v, acc_sh, sem_g, sem_s):
        cid = lax.axis_index("c")
        sid = lax.axis_index("s")
        wid = sid * NC + cid
        row0 = sid * RPT

        pltpu.sync_copy(zeros_hbm.at[pl.ds(row0, RPT)], acc_sh.at[pl.ds(row0, RPT)])
        pltpu.sync_copy(src_hbm.at[wid], src_v)
        pltpu.sync_copy(dst_hbm.at[wid], dst_v)
        plsc.subcore_barrier()

        pltpu.async_copy(table_hbm.at[src_v.at[0]], rows_v.at[0], sem_g)
        pltpu.async_copy(table_hbm.at[src_v.at[1]], rows_v.at[1], sem_g)

        def body(c, carry):
            buf = lax.rem(c, NBUF)
            pltpu.make_async_copy(table_hbm.at[src_v.at[c]],
                                  rows_v.at[buf], sem_g).wait()

            @pl.when(c + 2 < CPW)
            def _():
                pltpu.async_copy(table_hbm.at[src_v.at[c + 2]],
                                 rows_v.at[lax.rem(c + 2, NBUF)], sem_g)

            pltpu.sync_copy(rows_v.at[buf], acc_sh.at[dst_v.at[c]], add=True)
            return carry
        lax.fori_loop(0, CPW, body, 0)

        plsc.subcore_barrier()
        pltpu.sync_copy(acc_sh.at[pl.ds(row0, RPT)], out_hbm.at[cid, pl.ds(row0, RPT)])

    return sc_agg


_sc_agg80 = _make_sc_agg(W1)
_sc_agg64 = _make_sc_agg(W2)


def _t1_body(x_ref, w_ref, o_ref):
    y = jnp.dot(x_ref[...], w_ref[...], preferred_element_type=jnp.float32)
    col = lax.broadcasted_iota(jnp.int32, (RB, W1 - DH), 1)
    pad = jnp.where(col == 0, jnp.float32(1), jnp.float32(0))
    o_ref[...] = jnp.concatenate([y, pad], axis=1)


_t1 = pl.pallas_call(
    _t1_body,
    grid=(GRID,),
    in_specs=[
        pl.BlockSpec((RB, DF), lambda i: (i, 0)),
        pl.BlockSpec((DF, DH), lambda i: (0, 0)),
    ],
    out_specs=pl.BlockSpec((RB, W1), lambda i: (i, 0)),
    out_shape=jax.ShapeDtypeStruct((N, W1), jnp.float32),
)


def _c_body(acc_ref, x_ref, w1r_ref, b1_ref, wlin_ref, blin_ref,
            w2l_ref, w2r_ref, b2_ref, y2_ref, z2_ref, cnt_ref):
    acc = acc_ref[...]
    feat = acc[0, :, 0:DH] + acc[1, :, 0:DH]
    cnt = jnp.maximum(acc[0, :, DH] + acc[1, :, DH], 1.0)
    agg = feat / cnt[:, None]
    xb = x_ref[...]
    pre = agg + jnp.dot(xb, w1r_ref[...], preferred_element_type=jnp.float32) + b1_ref[...]
    nrm = jnp.sqrt(jnp.sum(pre * pre, axis=1, keepdims=True))
    hidden = jnp.maximum(pre / jnp.maximum(nrm, 1e-12), 0.0)
    wl = wlin_ref[...]
    h = jnp.maximum(
        jnp.dot(xb, wl[0:DF], preferred_element_type=jnp.float32)
        + jnp.dot(hidden, wl[DF:DF + DH], preferred_element_type=jnp.float32)
        + blin_ref[...],
        0.0,
    )
    y2_ref[...] = jnp.dot(h, w2l_ref[...], preferred_element_type=jnp.float32)
    z2_ref[...] = jnp.dot(h, w2r_ref[...], preferred_element_type=jnp.float32) + b2_ref[...]
    cnt_ref[...] = jnp.broadcast_to(cnt[:, None], (RB, 8))


_combine1 = pl.pallas_call(
    _c_body,
    grid=(GRID,),
    in_specs=[
        pl.BlockSpec((NC, RB, W1), lambda i: (0, i, 0)),
        pl.BlockSpec((RB, DF), lambda i: (i, 0)),
        pl.BlockSpec((DF, DH), lambda i: (0, 0)),
        pl.BlockSpec((1, DH), lambda i: (0, 0)),
        pl.BlockSpec((DF + DH, DH), lambda i: (0, 0)),
        pl.BlockSpec((1, DH), lambda i: (0, 0)),
        pl.BlockSpec((DH, DH), lambda i: (0, 0)),
        pl.BlockSpec((DH, DH), lambda i: (0, 0)),
        pl.BlockSpec((1, DH), lambda i: (0, 0)),
    ],
    out_specs=[
        pl.BlockSpec((RB, DH), lambda i: (i, 0)),
        pl.BlockSpec((RB, DH), lambda i: (i, 0)),
        pl.BlockSpec((RB, 8), lambda i: (i, 0)),
    ],
    out_shape=[
        jax.ShapeDtypeStruct((N, DH), jnp.float32),
        jax.ShapeDtypeStruct((N, DH), jnp.float32),
        jax.ShapeDtypeStruct((N, 8), jnp.float32),
    ],
)


def _e_body(acc_ref, z2_ref, cnt_ref, o_ref):
    acc = acc_ref[...]
    feat = acc[0] + acc[1]
    cnt = cnt_ref[...][:, 0]
    o = feat / cnt[:, None] + z2_ref[...]
    nrm = jnp.sqrt(jnp.sum(o * o, axis=1, keepdims=True))
    o_ref[...] = o / jnp.maximum(nrm, 1e-12)


_combine2 = pl.pallas_call(
    _e_body,
    grid=(GRID,),
    in_specs=[
        pl.BlockSpec((NC, RB, W2), lambda i: (0, i, 0)),
        pl.BlockSpec((RB, DH), lambda i: (i, 0)),
        pl.BlockSpec((RB, 8), lambda i: (i, 0)),
    ],
    out_specs=pl.BlockSpec((RB, DH), lambda i: (i, 0)),
    out_shape=jax.ShapeDtypeStruct((N, DH), jnp.float32),
)


def kernel(x, edge_index, W1_l, b1_l, W1_r, W_lin, b_lin, W2_l, b2_l, W2_r):
    ei = edge_index.astype(jnp.int32)
    ppw = EPW - E // NW
    pad_dst = jnp.broadcast_to(N + jnp.arange(ppw, dtype=jnp.int32)[None, :], (NW, ppw))
    src = jnp.pad(ei[0].reshape(NW, E // NW), ((0, 0), (0, ppw))).reshape(NW, CPW, CHUNK)
    dst = jnp.concatenate([ei[1].reshape(NW, E // NW), pad_dst], axis=1).reshape(NW, CPW, CHUNK)

    table1 = _t1(x, W1_l)
    acc1 = _sc_agg80(table1, src, dst, jnp.zeros((ROWS, W1), jnp.float32))
    y2, z2, cnt8 = _combine1(
        acc1, x, W1_r, b1_l.reshape(1, DH), W_lin, b_lin.reshape(1, DH),
        W2_l, W2_r, b2_l.reshape(1, DH),
    )
    acc2 = _sc_agg64(y2, src, dst, jnp.zeros((ROWS, W2), jnp.float32))
    return _combine2(acc2, z2, cnt8)


# speedup vs baseline: 1.6397x; 1.0373x over previous
import functools

import jax
import jax.numpy as jnp
from jax import lax
from jax.experimental import pallas as pl
from jax.experimental.pallas import tpu as pltpu
from jax.experimental.pallas import tpu_sc as plsc

N = 10000
E = 320000
DF = 128
DH = 64

NC, NS = 2, 16
NW = NC * NS
CHUNK = 128
NBUF = 4
CPW = 79
EPW = CPW * CHUNK
EP = NW * EPW
ROWS = 10112
RPT = ROWS // NS

W1 = 80
W2 = 64

RB = 2000
GRID = N // RB


def _make_sc_agg(width):
    mesh = plsc.VectorSubcoreMesh(
        core_axis_name="c", subcore_axis_name="s", num_cores=NC, num_subcores=NS
    )

    @functools.partial(
        pl.kernel,
        out_type=jax.ShapeDtypeStruct((NC, ROWS, width), jnp.float32),
        mesh=mesh,
        scratch_types=[
            pltpu.VMEM((CPW, CHUNK), jnp.int32),
            pltpu.VMEM((CPW, CHUNK), jnp.int32),
            pltpu.VMEM((NBUF, CHUNK, width), jnp.float32),
            pltpu.VMEM_SHARED((ROWS, width), jnp.float32),
            pltpu.SemaphoreType.DMA,
            pltpu.SemaphoreType.DMA,
        ],
        compiler_params=pltpu.CompilerParams(use_tc_tiling_on_sc=False),
    )
    def sc_agg(table_hbm, src_hbm, dst_hbm, zeros_hbm, out_hbm,
               src_v, dst_v, rows_v, acc_sh, sem_g, sem_s):
        cid = lax.axis_index("c")
        sid = lax.axis_index("s")
        wid = sid * NC + cid
        row0 = sid * RPT

        pltpu.sync_copy(zeros_hbm.at[pl.ds(row0, RPT)], acc_sh.at[pl.ds(row0, RPT)])
        pltpu.sync_copy(src_hbm.at[wid], src_v)
        pltpu.sync_copy(dst_hbm.at[wid], dst_v)
        plsc.subcore_barrier()

        for b in range(NBUF - 1):
            pltpu.async_copy(table_hbm.at[src_v.at[b]], rows_v.at[b], sem_g)

        def body(c, carry):
            buf = lax.rem(c, NBUF)
            pltpu.make_async_copy(table_hbm.at[src_v.at[c]],
                                  rows_v.at[buf], sem_g).wait()

            @pl.when(c + NBUF - 1 < CPW)
            def _():
                pltpu.async_copy(table_hbm.at[src_v.at[c + NBUF - 1]],
                                 rows_v.at[lax.rem(c + NBUF - 1, NBUF)], sem_g)

            pltpu.sync_copy(rows_v.at[buf], acc_sh.at[dst_v.at[c]], add=True)
            return carry
        lax.fori_loop(0, CPW, body, 0)

        plsc.subcore_barrier()
        pltpu.sync_copy(acc_sh.at[pl.ds(row0, RPT)], out_hbm.at[cid, pl.ds(row0, RPT)])

    return sc_agg


_sc_agg80 = _make_sc_agg(W1)
_sc_agg64 = _make_sc_agg(W2)


def _t1_body(x_ref, w_ref, o_ref):
    y = jnp.dot(x_ref[...], w_ref[...], preferred_element_type=jnp.float32)
    col = lax.broadcasted_iota(jnp.int32, (RB, W1 - DH), 1)
    pad = jnp.where(col == 0, jnp.float32(1), jnp.float32(0))
    o_ref[...] = jnp.concatenate([y, pad], axis=1)


_t1 = pl.pallas_call(
    _t1_body,
    grid=(GRID,),
    in_specs=[
        pl.BlockSpec((RB, DF), lambda i: (i, 0)),
        pl.BlockSpec((DF, DH), lambda i: (0, 0)),
    ],
    out_specs=pl.BlockSpec((RB, W1), lambda i: (i, 0)),
    out_shape=jax.ShapeDtypeStruct((N, W1), jnp.float32),
)


def _c_body(acc_ref, x_ref, w1r_ref, b1_ref, wlin_ref, blin_ref,
            w2l_ref, w2r_ref, b2_ref, y2_ref, z2_ref, cnt_ref):
    acc = acc_ref[...]
    feat = acc[0, :, 0:DH] + acc[1, :, 0:DH]
    cnt = jnp.maximum(acc[0, :, DH] + acc[1, :, DH], 1.0)
    agg = feat / cnt[:, None]
    xb = x_ref[...]
    pre = agg + jnp.dot(xb, w1r_ref[...], preferred_element_type=jnp.float32) + b1_ref[...]
    nrm = jnp.sqrt(jnp.sum(pre * pre, axis=1, keepdims=True))
    hidden = jnp.maximum(pre / jnp.maximum(nrm, 1e-12), 0.0)
    wl = wlin_ref[...]
    h = jnp.maximum(
        jnp.dot(xb, wl[0:DF], preferred_element_type=jnp.float32)
        + jnp.dot(hidden, wl[DF:DF + DH], preferred_element_type=jnp.float32)
        + blin_ref[...],
        0.0,
    )
    y2_ref[...] = jnp.dot(h, w2l_ref[...], preferred_element_type=jnp.float32)
    z2_ref[...] = jnp.dot(h, w2r_ref[...], preferred_element_type=jnp.float32) + b2_ref[...]
    cnt_ref[...] = jnp.broadcast_to(cnt[:, None], (RB, 8))


_combine1 = pl.pallas_call(
    _c_body,
    grid=(GRID,),
    in_specs=[
        pl.BlockSpec((NC, RB, W1), lambda i: (0, i, 0)),
        pl.BlockSpec((RB, DF), lambda i: (i, 0)),
        pl.BlockSpec((DF, DH), lambda i: (0, 0)),
        pl.BlockSpec((1, DH), lambda i: (0, 0)),
        pl.BlockSpec((DF + DH, DH), lambda i: (0, 0)),
        pl.BlockSpec((1, DH), lambda i: (0, 0)),
        pl.BlockSpec((DH, DH), lambda i: (0, 0)),
        pl.BlockSpec((DH, DH), lambda i: (0, 0)),
        pl.BlockSpec((1, DH), lambda i: (0, 0)),
    ],
    out_specs=[
        pl.BlockSpec((RB, DH), lambda i: (i, 0)),
        pl.BlockSpec((RB, DH), lambda i: (i, 0)),
        pl.BlockSpec((RB, 8), lambda i: (i, 0)),
    ],
    out_shape=[
        jax.ShapeDtypeStruct((N, DH), jnp.float32),
        jax.ShapeDtypeStruct((N, DH), jnp.float32),
        jax.ShapeDtypeStruct((N, 8), jnp.float32),
    ],
)


def _e_body(acc_ref, z2_ref, cnt_ref, o_ref):
    acc = acc_ref[...]
    feat = acc[0] + acc[1]
    cnt = cnt_ref[...][:, 0]
    o = feat / cnt[:, None] + z2_ref[...]
    nrm = jnp.sqrt(jnp.sum(o * o, axis=1, keepdims=True))
    o_ref[...] = o / jnp.maximum(nrm, 1e-12)


_combine2 = pl.pallas_call(
    _e_body,
    grid=(GRID,),
    in_specs=[
        pl.BlockSpec((NC, RB, W2), lambda i: (0, i, 0)),
        pl.BlockSpec((RB, DH), lambda i: (i, 0)),
        pl.BlockSpec((RB, 8), lambda i: (i, 0)),
    ],
    out_specs=pl.BlockSpec((RB, DH), lambda i: (i, 0)),
    out_shape=jax.ShapeDtypeStruct((N, DH), jnp.float32),
)


def kernel(x, edge_index, W1_l, b1_l, W1_r, W_lin, b_lin, W2_l, b2_l, W2_r):
    ei = edge_index.astype(jnp.int32)
    ppw = EPW - E // NW
    pad_dst = jnp.broadcast_to(N + jnp.arange(ppw, dtype=jnp.int32)[None, :], (NW, ppw))
    src = jnp.pad(ei[0].reshape(NW, E // NW), ((0, 0), (0, ppw))).reshape(NW, CPW, CHUNK)
    dst = jnp.concatenate([ei[1].reshape(NW, E // NW), pad_dst], axis=1).reshape(NW, CPW, CHUNK)

    table1 = _t1(x, W1_l)
    acc1 = _sc_agg80(table1, src, dst, jnp.zeros((ROWS, W1), jnp.float32))
    y2, z2, cnt8 = _combine1(
        acc1, x, W1_r, b1_l.reshape(1, DH), W_lin, b_lin.reshape(1, DH),
        W2_l, W2_r, b2_l.reshape(1, DH),
    )
    acc2 = _sc_agg64(y2, src, dst, jnp.zeros((ROWS, W2), jnp.float32))
    return _combine2(acc2, z2, cnt8)


# speedup vs baseline: 1.6420x; 1.0015x over previous
import functools

import jax
import jax.numpy as jnp
from jax import lax
from jax.experimental import pallas as pl
from jax.experimental.pallas import tpu as pltpu
from jax.experimental.pallas import tpu_sc as plsc

N = 10000
E = 320000
DF = 128
DH = 64

NC, NS = 2, 16
NW = NC * NS
CHUNK = 128
CPW = 79
EPW = CPW * CHUNK
EP = NW * EPW
ROWS = 10112
RPT = ROWS // NS

W1 = 80
W2 = 64

RB = 2000
GRID = N // RB


def _make_sc_agg(width, nbuf):
    NBUF = nbuf
    mesh = plsc.VectorSubcoreMesh(
        core_axis_name="c", subcore_axis_name="s", num_cores=NC, num_subcores=NS
    )

    @functools.partial(
        pl.kernel,
        out_type=jax.ShapeDtypeStruct((NC, ROWS, width), jnp.float32),
        mesh=mesh,
        scratch_types=[
            pltpu.VMEM((CPW, CHUNK), jnp.int32),
            pltpu.VMEM((CPW, CHUNK), jnp.int32),
            pltpu.VMEM((NBUF, CHUNK, width), jnp.float32),
            pltpu.VMEM_SHARED((ROWS, width), jnp.float32),
            pltpu.SemaphoreType.DMA,
            pltpu.SemaphoreType.DMA,
        ],
        compiler_params=pltpu.CompilerParams(use_tc_tiling_on_sc=False),
    )
    def sc_agg(table_hbm, src_hbm, dst_hbm, zeros_hbm, out_hbm,
               src_v, dst_v, rows_v, acc_sh, sem_g, sem_s):
        cid = lax.axis_index("c")
        sid = lax.axis_index("s")
        wid = sid * NC + cid
        row0 = sid * RPT

        pltpu.sync_copy(zeros_hbm.at[pl.ds(row0, RPT)], acc_sh.at[pl.ds(row0, RPT)])
        pltpu.sync_copy(src_hbm.at[wid], src_v)
        pltpu.sync_copy(dst_hbm.at[wid], dst_v)
        plsc.subcore_barrier()

        for b in range(NBUF - 1):
            pltpu.async_copy(table_hbm.at[src_v.at[b]], rows_v.at[b], sem_g)

        def body(c, carry):
            buf = lax.rem(c, NBUF)
            pltpu.make_async_copy(table_hbm.at[src_v.at[c]],
                                  rows_v.at[buf], sem_g).wait()

            @pl.when(c + NBUF - 1 < CPW)
            def _():
                pltpu.async_copy(table_hbm.at[src_v.at[c + NBUF - 1]],
                                 rows_v.at[lax.rem(c + NBUF - 1, NBUF)], sem_g)

            pltpu.sync_copy(rows_v.at[buf], acc_sh.at[dst_v.at[c]], add=True)
            return carry
        lax.fori_loop(0, CPW, body, 0)

        plsc.subcore_barrier()
        pltpu.sync_copy(acc_sh.at[pl.ds(row0, RPT)], out_hbm.at[cid, pl.ds(row0, RPT)])

    return sc_agg


_sc_agg80 = _make_sc_agg(W1, 4)
_sc_agg64 = _make_sc_agg(W2, 6)


def _t1_body(x_ref, w_ref, o_ref):
    y = jnp.dot(x_ref[...], w_ref[...], preferred_element_type=jnp.float32)
    col = lax.broadcasted_iota(jnp.int32, (RB, W1 - DH), 1)
    pad = jnp.where(col == 0, jnp.float32(1), jnp.float32(0))
    o_ref[...] = jnp.concatenate([y, pad], axis=1)


_t1 = pl.pallas_call(
    _t1_body,
    grid=(GRID,),
    in_specs=[
        pl.BlockSpec((RB, DF), lambda i: (i, 0)),
        pl.BlockSpec((DF, DH), lambda i: (0, 0)),
    ],
    out_specs=pl.BlockSpec((RB, W1), lambda i: (i, 0)),
    out_shape=jax.ShapeDtypeStruct((N, W1), jnp.float32),
)


def _c_body(acc_ref, x_ref, w1r_ref, b1_ref, wlin_ref, blin_ref,
            w2l_ref, w2r_ref, b2_ref, y2_ref, z2_ref, cnt_ref):
    acc = acc_ref[...]
    feat = acc[0, :, 0:DH] + acc[1, :, 0:DH]
    cnt = jnp.maximum(acc[0, :, DH] + acc[1, :, DH], 1.0)
    agg = feat / cnt[:, None]
    xb = x_ref[...]
    pre = agg + jnp.dot(xb, w1r_ref[...], preferred_element_type=jnp.float32) + b1_ref[...]
    nrm = jnp.sqrt(jnp.sum(pre * pre, axis=1, keepdims=True))
    hidden = jnp.maximum(pre / jnp.maximum(nrm, 1e-12), 0.0)
    wl = wlin_ref[...]
    h = jnp.maximum(
        jnp.dot(xb, wl[0:DF], preferred_element_type=jnp.float32)
        + jnp.dot(hidden, wl[DF:DF + DH], preferred_element_type=jnp.float32)
        + blin_ref[...],
        0.0,
    )
    y2_ref[...] = jnp.dot(h, w2l_ref[...], preferred_element_type=jnp.float32)
    z2_ref[...] = jnp.dot(h, w2r_ref[...], preferred_element_type=jnp.float32) + b2_ref[...]
    cnt_ref[...] = jnp.broadcast_to(cnt[:, None], (RB, 8))


_combine1 = pl.pallas_call(
    _c_body,
    grid=(GRID,),
    in_specs=[
        pl.BlockSpec((NC, RB, W1), lambda i: (0, i, 0)),
        pl.BlockSpec((RB, DF), lambda i: (i, 0)),
        pl.BlockSpec((DF, DH), lambda i: (0, 0)),
        pl.BlockSpec((1, DH), lambda i: (0, 0)),
        pl.BlockSpec((DF + DH, DH), lambda i: (0, 0)),
        pl.BlockSpec((1, DH), lambda i: (0, 0)),
        pl.BlockSpec((DH, DH), lambda i: (0, 0)),
        pl.BlockSpec((DH, DH), lambda i: (0, 0)),
        pl.BlockSpec((1, DH), lambda i: (0, 0)),
    ],
    out_specs=[
        pl.BlockSpec((RB, DH), lambda i: (i, 0)),
        pl.BlockSpec((RB, DH), lambda i: (i, 0)),
        pl.BlockSpec((RB, 8), lambda i: (i, 0)),
    ],
    out_shape=[
        jax.ShapeDtypeStruct((N, DH), jnp.float32),
        jax.ShapeDtypeStruct((N, DH), jnp.float32),
        jax.ShapeDtypeStruct((N, 8), jnp.float32),
    ],
)


def _e_body(acc_ref, z2_ref, cnt_ref, o_ref):
    acc = acc_ref[...]
    feat = acc[0] + acc[1]
    cnt = cnt_ref[...][:, 0]
    o = feat / cnt[:, None] + z2_ref[...]
    nrm = jnp.sqrt(jnp.sum(o * o, axis=1, keepdims=True))
    o_ref[...] = o / jnp.maximum(nrm, 1e-12)


_combine2 = pl.pallas_call(
    _e_body,
    grid=(GRID,),
    in_specs=[
        pl.BlockSpec((NC, RB, W2), lambda i: (0, i, 0)),
        pl.BlockSpec((RB, DH), lambda i: (i, 0)),
        pl.BlockSpec((RB, 8), lambda i: (i, 0)),
    ],
    out_specs=pl.BlockSpec((RB, DH), lambda i: (i, 0)),
    out_shape=jax.ShapeDtypeStruct((N, DH), jnp.float32),
)


def kernel(x, edge_index, W1_l, b1_l, W1_r, W_lin, b_lin, W2_l, b2_l, W2_r):
    ei = edge_index.astype(jnp.int32)
    ppw = EPW - E // NW
    pad_dst = jnp.broadcast_to(N + jnp.arange(ppw, dtype=jnp.int32)[None, :], (NW, ppw))
    src = jnp.pad(ei[0].reshape(NW, E // NW), ((0, 0), (0, ppw))).reshape(NW, CPW, CHUNK)
    dst = jnp.concatenate([ei[1].reshape(NW, E // NW), pad_dst], axis=1).reshape(NW, CPW, CHUNK)

    table1 = _t1(x, W1_l)
    acc1 = _sc_agg80(table1, src, dst, jnp.zeros((ROWS, W1), jnp.float32))
    y2, z2, cnt8 = _combine1(
        acc1, x, W1_r, b1_l.reshape(1, DH), W_lin, b_lin.reshape(1, DH),
        W2_l, W2_r, b2_l.reshape(1, DH),
    )
    acc2 = _sc_agg64(y2, src, dst, jnp.zeros((ROWS, W2), jnp.float32))
    return _combine2(acc2, z2, cnt8)
